# baseline (device time: 27917 ns/iter reference)
import jax
import jax.numpy as jnp
from jax import lax
from jax.experimental import pallas as pl
from jax.experimental.pallas import tpu as pltpu

N_DEV = 4
N_LAYERS = 3


def kernel(x, Win0, Wout0, Win1, Wout1, Win2, Wout2):
    b, d_local = x.shape
    h_dim = Win0.shape[1]

    def body(x_ref, win0_ref, wout0_ref, win1_ref, wout1_ref, win2_ref,
             wout2_ref, out_ref, mine_ref, comm_ref, send_sems, recv_sems):
        my_i = lax.axis_index("i")
        wins = [win0_ref, win1_ref, win2_ref]
        wouts = [wout0_ref, wout1_ref, wout2_ref]

        x_local = x_ref[...].astype(jnp.bfloat16)

        for k in range(N_LAYERS):
            p = k % 2
            partial = jnp.dot(
                x_local, wins[k][...].astype(jnp.bfloat16),
                preferred_element_type=jnp.float32,
            )
            mine_ref[p] = partial

            rdmas = []
            for off in range(1, N_DEV):
                peer = lax.rem(my_i + off, N_DEV)
                rdma = pltpu.make_async_remote_copy(
                    src_ref=mine_ref.at[p],
                    dst_ref=comm_ref.at[p, off - 1],
                    send_sem=send_sems.at[p, off - 1],
                    recv_sem=recv_sems.at[p, off - 1],
                    device_id=(peer,),
                    device_id_type=pl.DeviceIdType.MESH,
                )
                rdma.start()
                rdmas.append(rdma)
            for rdma in rdmas:
                rdma.wait()

            h = partial + comm_ref[p, 0] + comm_ref[p, 1] + comm_ref[p, 2]
            h = jnp.maximum(h, 0.0).astype(jnp.bfloat16)
            x_local = jnp.dot(
                h, wouts[k][...].astype(jnp.bfloat16),
                preferred_element_type=jnp.float32,
            )
            if k < N_LAYERS - 1:
                x_local = x_local.astype(jnp.bfloat16)

        out_ref[...] = x_local

    return pl.pallas_call(
        body,
        out_shape=jax.ShapeDtypeStruct((b, d_local), jnp.float32),
        in_specs=[pl.BlockSpec(memory_space=pltpu.VMEM)] * 7,
        out_specs=pl.BlockSpec(memory_space=pltpu.VMEM),
        scratch_shapes=[
            pltpu.VMEM((2, b, h_dim), jnp.float32),
            pltpu.VMEM((2, N_DEV - 1, b, h_dim), jnp.float32),
            pltpu.SemaphoreType.DMA((2, N_DEV - 1)),
            pltpu.SemaphoreType.DMA((2, N_DEV - 1)),
        ],
    )(x, Win0, Wout0, Win1, Wout1, Win2, Wout2)


# device time: 23959 ns/iter; 1.1652x vs baseline; 1.1652x over previous
import jax
import jax.numpy as jnp
from jax import lax
from jax.experimental import pallas as pl
from jax.experimental.pallas import tpu as pltpu

N_DEV = 4
N_LAYERS = 3


def kernel(x, Win0, Wout0, Win1, Wout1, Win2, Wout2):
    b, d_local = x.shape
    h_dim = Win0.shape[1]

    def body(x_ref, win0_ref, wout0_ref, win1_ref, wout1_ref, win2_ref,
             wout2_ref, out_ref, mine_ref, comm_ref, send_sems, recv_sems):
        my_i = lax.axis_index("i")
        wins = [win0_ref, win1_ref, win2_ref]
        wouts = [wout0_ref, wout1_ref, wout2_ref]

        x_local = x_ref[...].astype(jnp.bfloat16)

        inflight = {0: [], 1: []}
        for k in range(N_LAYERS):
            p = k % 2
            partial = jnp.dot(
                x_local, wins[k][...].astype(jnp.bfloat16),
                preferred_element_type=jnp.float32,
            )
            for rdma in inflight[p]:
                rdma.wait_send()
            inflight[p] = []
            mine_ref[p] = partial.astype(jnp.bfloat16)

            rdmas = []
            for off in range(1, N_DEV):
                peer = lax.rem(my_i + off, N_DEV)
                rdma = pltpu.make_async_remote_copy(
                    src_ref=mine_ref.at[p],
                    dst_ref=comm_ref.at[p, off - 1],
                    send_sem=send_sems.at[p, off - 1],
                    recv_sem=recv_sems.at[p, off - 1],
                    device_id=(peer,),
                    device_id_type=pl.DeviceIdType.MESH,
                )
                rdma.start()
                rdmas.append(rdma)
            inflight[p] = rdmas
            for rdma in rdmas:
                rdma.wait_recv()

            h = (partial
                 + comm_ref[p, 0].astype(jnp.float32)
                 + comm_ref[p, 1].astype(jnp.float32)
                 + comm_ref[p, 2].astype(jnp.float32))
            h = jnp.maximum(h, 0.0).astype(jnp.bfloat16)
            x_local = jnp.dot(
                h, wouts[k][...].astype(jnp.bfloat16),
                preferred_element_type=jnp.float32,
            )
            if k < N_LAYERS - 1:
                x_local = x_local.astype(jnp.bfloat16)

        for rdmas in inflight.values():
            for rdma in rdmas:
                rdma.wait_send()

        out_ref[...] = x_local

    return pl.pallas_call(
        body,
        out_shape=jax.ShapeDtypeStruct((b, d_local), jnp.float32),
        in_specs=[pl.BlockSpec(memory_space=pltpu.VMEM)] * 7,
        out_specs=pl.BlockSpec(memory_space=pltpu.VMEM),
        scratch_shapes=[
            pltpu.VMEM((2, b, h_dim), jnp.bfloat16),
            pltpu.VMEM((2, N_DEV - 1, b, h_dim), jnp.bfloat16),
            pltpu.SemaphoreType.DMA((2, N_DEV - 1)),
            pltpu.SemaphoreType.DMA((2, N_DEV - 1)),
        ],
    )(x, Win0, Wout0, Win1, Wout1, Win2, Wout2)


# device time: 21296 ns/iter; 1.3109x vs baseline; 1.1250x over previous
import jax
import jax.numpy as jnp
from jax import lax
from jax.experimental import pallas as pl
from jax.experimental.pallas import tpu as pltpu

N_DEV = 4
N_LAYERS = 3


def kernel(x, Win0, Wout0, Win1, Wout1, Win2, Wout2):
    b, d_local = x.shape
    h_dim = Win0.shape[1]

    def body(x_ref, win0_ref, wout0_ref, win1_ref, wout1_ref, win2_ref,
             wout2_ref, out_ref, mine_ref, comm_ref, send_sems, recv_sems):
        my_i = lax.axis_index("i")
        wins = [win0_ref, win1_ref, win2_ref]
        wouts = [wout0_ref, wout1_ref, wout2_ref]

        barrier_sem = pltpu.get_barrier_semaphore()
        for off in range(1, N_DEV):
            pl.semaphore_signal(
                barrier_sem, inc=1,
                device_id=(lax.rem(my_i + off, N_DEV),),
                device_id_type=pl.DeviceIdType.MESH,
            )
        pl.semaphore_wait(barrier_sem, N_DEV - 1)

        x_local = x_ref[...].astype(jnp.bfloat16)

        inflight = {0: [], 1: []}
        for k in range(N_LAYERS):
            p = k % 2
            partial = jnp.dot(
                x_local, wins[k][...].astype(jnp.bfloat16),
                preferred_element_type=jnp.float32,
            )
            for rdma in inflight[p]:
                rdma.wait_send()
            inflight[p] = []
            mine_ref[p] = partial.astype(jnp.bfloat16)

            rdmas = []
            for off in range(1, N_DEV):
                peer = lax.rem(my_i + off, N_DEV)
                rdma = pltpu.make_async_remote_copy(
                    src_ref=mine_ref.at[p],
                    dst_ref=comm_ref.at[p, off - 1],
                    send_sem=send_sems.at[p, off - 1],
                    recv_sem=recv_sems.at[p, off - 1],
                    device_id=(peer,),
                    device_id_type=pl.DeviceIdType.MESH,
                )
                rdma.start()
                rdmas.append(rdma)
            inflight[p] = rdmas
            for rdma in rdmas:
                rdma.wait_recv()

            h = (partial
                 + comm_ref[p, 0].astype(jnp.float32)
                 + comm_ref[p, 1].astype(jnp.float32)
                 + comm_ref[p, 2].astype(jnp.float32))
            h = jnp.maximum(h, 0.0).astype(jnp.bfloat16)
            x_local = jnp.dot(
                h, wouts[k][...].astype(jnp.bfloat16),
                preferred_element_type=jnp.float32,
            )
            if k < N_LAYERS - 1:
                x_local = x_local.astype(jnp.bfloat16)

        for rdmas in inflight.values():
            for rdma in rdmas:
                rdma.wait_send()

        out_ref[...] = x_local

    return pl.pallas_call(
        body,
        out_shape=jax.ShapeDtypeStruct((b, d_local), jnp.float32),
        in_specs=[pl.BlockSpec(memory_space=pltpu.VMEM)] * 7,
        out_specs=pl.BlockSpec(memory_space=pltpu.VMEM),
        scratch_shapes=[
            pltpu.VMEM((2, b, h_dim), jnp.bfloat16),
            pltpu.VMEM((2, N_DEV - 1, b, h_dim), jnp.bfloat16),
            pltpu.SemaphoreType.DMA((2, N_DEV - 1)),
            pltpu.SemaphoreType.DMA((2, N_DEV - 1)),
        ],
        compiler_params=pltpu.CompilerParams(collective_id=0),
    )(x, Win0, Wout0, Win1, Wout1, Win2, Wout2)
